# baseline (speedup 1.0000x reference)
import functools

import jax
import jax.numpy as jnp
from jax import lax
from jax.experimental import pallas as pl
from jax.experimental.pallas import tpu as pltpu
from jax.experimental.pallas import tpu_sc as plsc

_B, _I, _E = 16384, 128, 18
_LO, _HI = 64, 71
_NW = 32
_RW = _B // _NW
_HC = _RW // 4


def _sc_body(x_hbm, t_hbm, out_hbm, img0, img1, xpf, carr, obak, tvm,
             sem0, sem1, sempf):
    wid = lax.axis_index("s") * 2 + lax.axis_index("c")
    pltpu.sync_copy(t_hbm, tvm)
    lane = lax.iota(jnp.int32, 16)
    lmask = lane < (_HI - _LO)
    imgs = (img0, img1)
    sems = (sem0, sem1)

    row_base = wid * _RW
    pltpu.sync_copy(x_hbm.at[pl.ds(row_base, _HC), :], xpf)
    pf = None
    for half in range(4):
        row0 = row_base + half * _HC
        if pf is not None:
            pf.wait()

        def _snap(r, _):
            v = xpf[r, pl.ds(_LO, 16)]
            carr[r, :] = v.astype(jnp.int32)
            obak[r, :] = v
            for q in (0, 1, 2, 3, 5, 6, 7):
                w = xpf[r, pl.ds(q * 16, 16)]
                img0[r, pl.ds(q * 16, 16)] = w
                img1[r, pl.ds(q * 16, 16)] = w
            return 0

        lax.fori_loop(0, _HC, _snap, 0)
        if half < 3:
            pf = pltpu.async_copy(
                x_hbm.at[pl.ds(row0 + _HC, _HC), :], xpf, sempf)

        pending = [None, None]
        for e in range(_E):
            buf = imgs[e % 2]
            if pending[e % 2] is not None:
                pending[e % 2].wait()

            def _patch(r, _, _eoff=e * 52, _buf=buf):
                idx = carr[r, :] + _eoff
                g = plsc.load_gather(tvm, [idx])
                merged = jnp.where(lmask, g, obak[r, :])
                _buf[r, pl.ds(_LO, 16)] = merged
                return 0

            lax.fori_loop(0, _HC, _patch, 0)
            pending[e % 2] = pltpu.async_copy(
                buf, out_hbm.at[e, pl.ds(row0, _HC), :], sems[e % 2])
        pending[0].wait()
        pending[1].wait()


@jax.jit
def kernel(x, card_buffer):
    if x.ndim == 3:
        x = x[:, 0, :]
    B = x.shape[0]
    f32 = jnp.float32
    T = jnp.concatenate([card_buffer.T.reshape(-1), jnp.zeros((1024 - 52 * _E,), f32)])

    mesh = plsc.VectorSubcoreMesh(core_axis_name="c", subcore_axis_name="s")
    run = functools.partial(
        pl.kernel,
        mesh=mesh,
        compiler_params=pltpu.CompilerParams(needs_layout_passes=False),
        out_type=jax.ShapeDtypeStruct((_E, B, _I), f32),
        scratch_types=[
            pltpu.VMEM((_HC, _I), f32),
            pltpu.VMEM((_HC, _I), f32),
            pltpu.VMEM((_HC, _I), f32),
            pltpu.VMEM((_HC, 16), jnp.int32),
            pltpu.VMEM((_HC, 16), f32),
            pltpu.VMEM((1024,), f32),
            pltpu.SemaphoreType.DMA,
            pltpu.SemaphoreType.DMA,
            pltpu.SemaphoreType.DMA,
        ],
    )(_sc_body)
    out3 = run(x, T)
    return jnp.transpose(out3, (1, 2, 0))
